# baseline (speedup 1.0000x reference)
import jax
import jax.numpy as jnp
from jax.experimental import pallas as pl
from jax.experimental.pallas import tpu as pltpu

_NUM_CLASSES = 3
_THRESHOLD = 0.35
_NEG_POS_RATIO = 7
_V0 = 0.1
_V1 = 0.2
_P = 16800
_ROWS = 132
_LANES = 128
_PAD = _ROWS * _LANES
_NOBJ = 8
_MAX_FINITE_BITS = 0x7F7FFFFF


def _smooth_l1(x, t):
    d = jnp.abs(x - t)
    return jnp.where(d < 1.0, 0.5 * d * d, d - 0.5)


def _mbl_kernel(tgt_ref, pri_ref, loc_ref, conf_ref, lnd_ref,
                o_l, o_pce, o_lr, o_lm, o_np):
    f32 = jnp.float32
    tg = tgt_ref[0]
    px, py, pw, ph = pri_ref[0, 0], pri_ref[0, 1], pri_ref[0, 2], pri_ref[0, 3]
    ax1 = px - pw * 0.5
    ay1 = py - ph * 0.5
    ax2 = px + pw * 0.5
    ay2 = py + ph * 0.5
    area_p = (ax2 - ax1) * (ay2 - ay1)

    rows = jax.lax.broadcasted_iota(jnp.int32, (_ROWS, _LANES), 0)
    cols = jax.lax.broadcasted_iota(jnp.int32, (_ROWS, _LANES), 1)
    lidx = rows * _LANES + cols
    is_pad = lidx >= _P

    ovs = []
    bpo = []
    bpi = []
    for j in range(_NOBJ):
        tx1, ty1, tx2, ty2 = tg[j, 0], tg[j, 1], tg[j, 2], tg[j, 3]
        iw = jnp.maximum(jnp.minimum(tx2, ax2) - jnp.maximum(tx1, ax1), 0.0)
        ih = jnp.maximum(jnp.minimum(ty2, ay2) - jnp.maximum(ty1, ay1), 0.0)
        inter = iw * ih
        area_t = (tx2 - tx1) * (ty2 - ty1)
        ov = inter / (area_t + area_p - inter)
        ovs.append(ov)
        m = jnp.max(ov)
        bpo.append(m)
        bpi.append(jnp.min(jnp.where(ov == m, lidx, jnp.int32(0x7FFFFFFF))))

    items = [(ovs[j], jnp.int32(j)) for j in range(_NOBJ)]
    while len(items) > 1:
        nxt = []
        for a in range(0, len(items), 2):
            (ovl, il), (ovr, ir) = items[a], items[a + 1]
            take_r = ovr > ovl
            nxt.append((jnp.maximum(ovl, ovr), jnp.where(take_r, ir, il)))
        items = nxt
    bto, bti = items[0]

    valid = [bpo[j] >= 0.2 for j in range(_NOBJ)]
    any_valid = valid[0]
    for j in range(1, _NOBJ):
        any_valid = jnp.logical_or(any_valid, valid[j])

    sc = [(lidx == bpi[j], jnp.int32(j),
           jnp.where(valid[j], jnp.float32(1.0), jnp.float32(0.0)))
          for j in range(_NOBJ)]
    while len(sc) > 1:
        nxt = []
        for a in range(0, len(sc), 2):
            (ml, jl, vl), (mr, jr, vr) = sc[a], sc[a + 1]
            nxt.append((jnp.logical_or(ml, mr),
                        jnp.where(mr, jr, jl),
                        jnp.where(mr, vr, vl)))
        sc = nxt
    sm, sj, sv = sc[0]
    bti = jnp.where(sm, sj, bti)
    bto = jnp.where(jnp.logical_and(sm, sv > 0.5), 2.0, bto)

    w = [bti == j for j in range(_NOBJ)]
    wf = [w[j].astype(f32) for j in range(_NOBJ)]

    def _tree_sum(terms):
        while len(terms) > 1:
            nxt = [terms[a] + terms[a + 1]
                   for a in range(0, len(terms) - 1, 2)]
            if len(terms) % 2:
                nxt.append(terms[-1])
            terms = nxt
        return terms[0]

    conf = _tree_sum([wf[j] * tg[j, 14] for j in range(_NOBJ)])
    conf = jnp.where(bto < _THRESHOLD, 0.0, conf)
    conf = jnp.where(any_valid, conf, 0.0)
    pos = conf > 0.0
    pos_f = pos.astype(f32)

    def sel(col):
        return _tree_sum([wf[j] * tg[j, col] for j in range(_NOBJ)])

    mx1, my1, mx2, my2 = sel(0), sel(1), sel(2), sel(3)
    g_cx = ((mx1 + mx2) * 0.5 - px) / (_V0 * pw)
    g_cy = ((my1 + my2) * 0.5 - py) / (_V0 * ph)
    g_w = jnp.log((mx2 - mx1) / pw) / _V1
    g_h = jnp.log((my2 - my1) / ph) / _V1
    loss_l = jnp.sum(_tree_sum(
        [pos_f * _smooth_l1(loc_ref[0, 0], g_cx),
         pos_f * _smooth_l1(loc_ref[0, 1], g_cy),
         pos_f * _smooth_l1(loc_ref[0, 2], g_w),
         pos_f * _smooth_l1(loc_ref[0, 3], g_h)]))

    face_f = (conf == 1.0).astype(f32)
    lm_terms = []
    for c in range(10):
        ml = sel(4 + c)
        if c % 2 == 0:
            g = (ml - px) / (_V0 * pw)
        else:
            g = (ml - py) / (_V0 * ph)
        wt = pos_f if c < 4 else face_f
        lm_terms.append(wt * _smooth_l1(lnd_ref[0, c], g))
    loss_lm = jnp.sum(_tree_sum(lm_terms))

    c0, c1, c2 = conf_ref[0, 0], conf_ref[0, 1], conf_ref[0, 2]
    mx = jnp.maximum(c0, jnp.maximum(c1, c2))
    lse = mx + jnp.log(jnp.exp(c0 - mx) + jnp.exp(c1 - mx) + jnp.exp(c2 - mx))
    gth = jnp.where(conf == 1.0, c1, jnp.where(conf == 2.0, c2, c0))
    ce = lse - gth
    s_pce = jnp.sum(pos_f * ce)

    lr = jnp.where(jnp.logical_or(pos, is_pad), 0.0, ce)
    npos_f = jnp.sum(pos_f)

    o_l[0] = jnp.full((1, _LANES), loss_l)
    o_pce[0] = jnp.full((1, _LANES), s_pce)
    o_lr[0] = lr
    o_lm[0] = jnp.full((1, _LANES), loss_lm)
    o_np[0] = jnp.full((1, _LANES), npos_f)


def _topk_kernel(lr_ref, l_ref, pce_ref, lm_ref, np_ref, o_l, o_c, o_lm):
    f32 = jnp.float32
    lr = lr_ref[...]
    num = lr.shape[0]
    bits = jax.lax.bitcast_convert_type(lr, jnp.int32)
    npos = jnp.sum(np_ref[...], axis=(1, 2)) * (1.0 / _LANES)
    kf = jnp.minimum(_NEG_POS_RATIO * npos, float(_P - 1))

    def body(_, carry):
        lo, hi = carry
        mid = lo + (hi - lo + 1) // 2
        ge = (bits >= mid[:, None, None]).astype(f32)
        cnt = jnp.sum(jnp.sum(ge, axis=1), axis=1)
        ok = cnt >= kf
        return jnp.where(ok, mid, lo), jnp.where(ok, hi, mid - 1)

    lo, _ = jax.lax.fori_loop(
        0, 31, body,
        (jnp.zeros((num,), jnp.int32),
         jnp.full((num,), _MAX_FINITE_BITS, jnp.int32)))
    tval = jax.lax.bitcast_convert_type(lo, f32)
    gt = bits > lo[:, None, None]
    cnt_gt = jnp.sum(jnp.sum(gt.astype(f32), axis=1), axis=1)
    sum_gt = jnp.sum(jnp.sum(jnp.where(gt, lr, 0.0), axis=1), axis=1)
    s_tk = sum_gt + (kf - cnt_gt) * tval

    n = jnp.maximum(jnp.sum(npos), 1.0)
    loss_l = jnp.sum(l_ref[...][..., 0:1]) / n
    loss_c = (jnp.sum(pce_ref[...][..., 0:1]) + jnp.sum(s_tk)) / n
    loss_lm = jnp.sum(lm_ref[...][..., 0:1]) / n
    o_l[...] = jnp.full((8, _LANES), loss_l)
    o_c[...] = jnp.full((8, _LANES), loss_c)
    o_lm[...] = jnp.full((8, _LANES), loss_lm)


def _prep(x):
    num, _, ch = x.shape
    x = jnp.transpose(x, (0, 2, 1))
    x = jnp.pad(x, ((0, 0), (0, 0), (0, _PAD - _P)))
    return x.reshape(num, ch, _ROWS, _LANES)


@jax.jit
def kernel(loc_data, conf_data, landm_data, priors, targets):
    num = loc_data.shape[0]
    loc_r = _prep(loc_data)
    conf_r = _prep(conf_data)
    lnd_r = _prep(landm_data)
    pri_t = jnp.transpose(priors)
    pad_col = jnp.broadcast_to(
        jnp.array([[2.0], [2.0], [0.1], [0.1]], jnp.float32), (4, _PAD - _P))
    pri_r = jnp.concatenate([pri_t, pad_col], axis=1).reshape(1, 4, _ROWS, _LANES)

    part_sds = jax.ShapeDtypeStruct((num, 1, _LANES), jnp.float32)
    out_sds = [part_sds, part_sds,
               jax.ShapeDtypeStruct((num, _ROWS, _LANES), jnp.float32),
               part_sds, part_sds]
    part_spec = pl.BlockSpec((1, 1, _LANES), lambda i: (i, 0, 0))
    s_l, s_pce, lr, s_lm, s_np = pl.pallas_call(
        _mbl_kernel,
        grid=(num,),
        in_specs=[
            pl.BlockSpec((1, _NOBJ, 15), lambda i: (i, 0, 0)),
            pl.BlockSpec((1, 4, _ROWS, _LANES), lambda i: (0, 0, 0, 0)),
            pl.BlockSpec((1, 4, _ROWS, _LANES), lambda i: (i, 0, 0, 0)),
            pl.BlockSpec((1, 3, _ROWS, _LANES), lambda i: (i, 0, 0, 0)),
            pl.BlockSpec((1, 10, _ROWS, _LANES), lambda i: (i, 0, 0, 0)),
        ],
        out_specs=[part_spec, part_spec,
                   pl.BlockSpec((1, _ROWS, _LANES), lambda i: (i, 0, 0)),
                   part_spec, part_spec],
        out_shape=out_sds,
        compiler_params=pltpu.CompilerParams(
            dimension_semantics=("parallel",),
            allow_input_fusion=[False, False, True, True, True]),
    )(targets, pri_r, loc_r, conf_r, lnd_r)

    sc_sds = jax.ShapeDtypeStruct((8, _LANES), jnp.float32)
    o_l, o_c, o_lm = pl.pallas_call(
        _topk_kernel,
        out_shape=[sc_sds, sc_sds, sc_sds],
    )(lr, s_l, s_pce, s_lm, s_np)
    return o_l[0, 0], o_c[0, 0], o_lm[0, 0]


# speedup vs baseline: 1.0014x; 1.0014x over previous
import jax
import jax.numpy as jnp
from jax.experimental import pallas as pl
from jax.experimental.pallas import tpu as pltpu

_NUM_CLASSES = 3
_THRESHOLD = 0.35
_NEG_POS_RATIO = 7
_V0 = 0.1
_V1 = 0.2
_P = 16800
_ROWS = 132
_LANES = 128
_PAD = _ROWS * _LANES
_NOBJ = 8
_MAX_FINITE_BITS = 0x7F7FFFFF


def _smooth_l1(x, t):
    d = jnp.abs(x - t)
    return jnp.where(d < 1.0, 0.5 * d * d, d - 0.5)


def _mbl_kernel(tgt_ref, pri_ref, loc_ref, conf_ref, lnd_ref,
                o_l, o_pce, o_lr, o_lm, o_np):
    f32 = jnp.float32
    tg = tgt_ref[0]
    px, py, pw, ph = pri_ref[0, 0], pri_ref[0, 1], pri_ref[0, 2], pri_ref[0, 3]
    ax1 = px - pw * 0.5
    ay1 = py - ph * 0.5
    ax2 = px + pw * 0.5
    ay2 = py + ph * 0.5
    area_p = (ax2 - ax1) * (ay2 - ay1)

    rows = jax.lax.broadcasted_iota(jnp.int32, (_ROWS, _LANES), 0)
    cols = jax.lax.broadcasted_iota(jnp.int32, (_ROWS, _LANES), 1)
    lidx = rows * _LANES + cols
    is_pad = lidx >= _P

    ovs = []
    bpo = []
    bpi = []
    for j in range(_NOBJ):
        tx1, ty1, tx2, ty2 = tg[j, 0], tg[j, 1], tg[j, 2], tg[j, 3]
        iw = jnp.maximum(jnp.minimum(tx2, ax2) - jnp.maximum(tx1, ax1), 0.0)
        ih = jnp.maximum(jnp.minimum(ty2, ay2) - jnp.maximum(ty1, ay1), 0.0)
        inter = iw * ih
        area_t = (tx2 - tx1) * (ty2 - ty1)
        ov = inter / (area_t + area_p - inter)
        ovs.append(ov)
        m = jnp.max(ov)
        bpo.append(m)
        bpi.append(jnp.min(jnp.where(ov == m, lidx, jnp.int32(0x7FFFFFFF))))

    items = [(ovs[j], jnp.int32(j)) for j in range(_NOBJ)]
    while len(items) > 1:
        nxt = []
        for a in range(0, len(items), 2):
            (ovl, il), (ovr, ir) = items[a], items[a + 1]
            take_r = ovr > ovl
            nxt.append((jnp.maximum(ovl, ovr), jnp.where(take_r, ir, il)))
        items = nxt
    bto, bti = items[0]

    valid = [bpo[j] >= 0.2 for j in range(_NOBJ)]
    any_valid = valid[0]
    for j in range(1, _NOBJ):
        any_valid = jnp.logical_or(any_valid, valid[j])

    sc = [(lidx == bpi[j], jnp.int32(j),
           jnp.where(valid[j], jnp.float32(1.0), jnp.float32(0.0)))
          for j in range(_NOBJ)]
    while len(sc) > 1:
        nxt = []
        for a in range(0, len(sc), 2):
            (ml, jl, vl), (mr, jr, vr) = sc[a], sc[a + 1]
            nxt.append((jnp.logical_or(ml, mr),
                        jnp.where(mr, jr, jl),
                        jnp.where(mr, vr, vl)))
        sc = nxt
    sm, sj, sv = sc[0]
    bti = jnp.where(sm, sj, bti)
    bto = jnp.where(jnp.logical_and(sm, sv > 0.5), 2.0, bto)

    w = [bti == j for j in range(_NOBJ)]
    wf = [w[j].astype(f32) for j in range(_NOBJ)]

    def _tree_sum(terms):
        while len(terms) > 1:
            nxt = [terms[a] + terms[a + 1]
                   for a in range(0, len(terms) - 1, 2)]
            if len(terms) % 2:
                nxt.append(terms[-1])
            terms = nxt
        return terms[0]

    conf = _tree_sum([wf[j] * tg[j, 14] for j in range(_NOBJ)])
    conf = jnp.where(bto < _THRESHOLD, 0.0, conf)
    conf = jnp.where(any_valid, conf, 0.0)
    pos = conf > 0.0
    pos_f = pos.astype(f32)

    def sel(col):
        return _tree_sum([wf[j] * tg[j, col] for j in range(_NOBJ)])

    mx1, my1, mx2, my2 = sel(0), sel(1), sel(2), sel(3)
    g_cx = ((mx1 + mx2) * 0.5 - px) / (_V0 * pw)
    g_cy = ((my1 + my2) * 0.5 - py) / (_V0 * ph)
    g_w = jnp.log((mx2 - mx1) / pw) / _V1
    g_h = jnp.log((my2 - my1) / ph) / _V1
    loss_l = jnp.sum(_tree_sum(
        [pos_f * _smooth_l1(loc_ref[0, 0], g_cx),
         pos_f * _smooth_l1(loc_ref[0, 1], g_cy),
         pos_f * _smooth_l1(loc_ref[0, 2], g_w),
         pos_f * _smooth_l1(loc_ref[0, 3], g_h)]))

    face_f = (conf == 1.0).astype(f32)
    lm_terms = []
    for c in range(10):
        ml = sel(4 + c)
        if c % 2 == 0:
            g = (ml - px) / (_V0 * pw)
        else:
            g = (ml - py) / (_V0 * ph)
        wt = pos_f if c < 4 else face_f
        lm_terms.append(wt * _smooth_l1(lnd_ref[0, c], g))
    loss_lm = jnp.sum(_tree_sum(lm_terms))

    c0, c1, c2 = conf_ref[0, 0], conf_ref[0, 1], conf_ref[0, 2]
    mx = jnp.maximum(c0, jnp.maximum(c1, c2))
    lse = mx + jnp.log(jnp.exp(c0 - mx) + jnp.exp(c1 - mx) + jnp.exp(c2 - mx))
    gth = jnp.where(conf == 1.0, c1, jnp.where(conf == 2.0, c2, c0))
    ce = lse - gth
    s_pce = jnp.sum(pos_f * ce)

    lr = jnp.where(jnp.logical_or(pos, is_pad), 0.0, ce)
    npos_f = jnp.sum(pos_f)

    o_l[0] = jnp.full((1, _LANES), loss_l)
    o_pce[0] = jnp.full((1, _LANES), s_pce)
    o_lr[0] = lr
    o_lm[0] = jnp.full((1, _LANES), loss_lm)
    o_np[0] = jnp.full((1, _LANES), npos_f)


def _topk_kernel(lr_ref, l_ref, pce_ref, lm_ref, np_ref, o_l, o_c, o_lm):
    f32 = jnp.float32
    lr = lr_ref[...]
    num = lr.shape[0]
    bits = jax.lax.bitcast_convert_type(lr, jnp.int32)
    npos = jnp.sum(np_ref[...], axis=(1, 2)) * (1.0 / _LANES)
    kf = jnp.minimum(_NEG_POS_RATIO * npos, float(_P - 1))

    def body(_, carry):
        lo, hi = carry
        mid = lo + (hi - lo + 1) // 2
        ge = (bits >= mid[:, None, None]).astype(f32)
        cnt = jnp.sum(jnp.sum(ge, axis=1), axis=1)
        ok = cnt >= kf
        return jnp.where(ok, mid, lo), jnp.where(ok, hi, mid - 1)

    lo, _ = jax.lax.fori_loop(
        0, 31, body,
        (jnp.zeros((num,), jnp.int32),
         jnp.full((num,), _MAX_FINITE_BITS, jnp.int32)))
    tval = jax.lax.bitcast_convert_type(lo, f32)
    gt = bits > lo[:, None, None]
    cnt_gt = jnp.sum(jnp.sum(gt.astype(f32), axis=1), axis=1)
    sum_gt = jnp.sum(jnp.sum(jnp.where(gt, lr, 0.0), axis=1), axis=1)
    s_tk = sum_gt + (kf - cnt_gt) * tval

    n = jnp.maximum(jnp.sum(npos), 1.0)
    loss_l = jnp.sum(l_ref[...][..., 0:1]) / n
    loss_c = (jnp.sum(pce_ref[...][..., 0:1]) + jnp.sum(s_tk)) / n
    loss_lm = jnp.sum(lm_ref[...][..., 0:1]) / n
    o_l[...] = jnp.full((8, _LANES), loss_l)
    o_c[...] = jnp.full((8, _LANES), loss_c)
    o_lm[...] = jnp.full((8, _LANES), loss_lm)


def _prep(x):
    num, _, ch = x.shape
    x = jnp.transpose(x, (0, 2, 1))
    x = jnp.pad(x, ((0, 0), (0, 0), (0, _PAD - _P)))
    return x.reshape(num, ch, _ROWS, _LANES)


@jax.jit
def kernel(loc_data, conf_data, landm_data, priors, targets):
    num = loc_data.shape[0]
    loc_r = _prep(loc_data)
    conf_r = _prep(conf_data)
    lnd_r = _prep(landm_data)
    pri_t = jnp.transpose(priors)
    pad_col = jnp.broadcast_to(
        jnp.array([[2.0], [2.0], [0.1], [0.1]], jnp.float32), (4, _PAD - _P))
    pri_r = jnp.concatenate([pri_t, pad_col], axis=1).reshape(1, 4, _ROWS, _LANES)

    part_sds = jax.ShapeDtypeStruct((num, 1, _LANES), jnp.float32)
    out_sds = [part_sds, part_sds,
               jax.ShapeDtypeStruct((num, _ROWS, _LANES), jnp.float32),
               part_sds, part_sds]
    part_spec = pl.BlockSpec((1, 1, _LANES), lambda i: (i, 0, 0))
    s_l, s_pce, lr, s_lm, s_np = pl.pallas_call(
        _mbl_kernel,
        grid=(num,),
        in_specs=[
            pl.BlockSpec((1, _NOBJ, 15), lambda i: (i, 0, 0)),
            pl.BlockSpec((1, 4, _ROWS, _LANES), lambda i: (0, 0, 0, 0)),
            pl.BlockSpec((1, 4, _ROWS, _LANES), lambda i: (i, 0, 0, 0)),
            pl.BlockSpec((1, 3, _ROWS, _LANES), lambda i: (i, 0, 0, 0)),
            pl.BlockSpec((1, 10, _ROWS, _LANES), lambda i: (i, 0, 0, 0)),
        ],
        out_specs=[part_spec, part_spec,
                   pl.BlockSpec((1, _ROWS, _LANES), lambda i: (i, 0, 0)),
                   part_spec, part_spec],
        out_shape=out_sds,
        compiler_params=pltpu.CompilerParams(
            dimension_semantics=("parallel",)),
    )(targets, pri_r, loc_r, conf_r, lnd_r)

    sc_sds = jax.ShapeDtypeStruct((8, _LANES), jnp.float32)
    o_l, o_c, o_lm = pl.pallas_call(
        _topk_kernel,
        out_shape=[sc_sds, sc_sds, sc_sds],
    )(lr, s_l, s_pce, s_lm, s_np)
    return o_l[0, 0], o_c[0, 0], o_lm[0, 0]
